# Initial kernel scaffold; baseline (speedup 1.0000x reference)
import jax
import jax.numpy as jnp
from jax.experimental import pallas as pl


def kernel(x, edge_index, batch, params):
    raise NotImplementedError("write your pallas kernel here")


# speedup vs baseline: 1.0000x; 1.0000x over previous
import jax
import jax.numpy as jnp
from jax.experimental import pallas as pl
from jax.experimental.pallas import tpu as pltpu

N = 10000
E = 320000
D = 128
H = 256
LAT = 64
L = 3
G = 64

_BN_SCALE = 1.0 / (1.0 + 1e-5) ** 0.5


def _bn(x, g, b):
    return x * (_BN_SCALE * g) + b


def _head_body(pooled_ref, w1_ref, b1_ref, lng_ref, lnb_ref, w2_ref, b2_ref,
               out_ref):
    pooled = pooled_ref[...]
    z = jnp.dot(pooled, w1_ref[...], preferred_element_type=jnp.float32)
    z = z + b1_ref[...]
    m = jnp.mean(z, axis=-1, keepdims=True)
    v = jnp.mean((z - m) ** 2, axis=-1, keepdims=True)
    z = (z - m) * jax.lax.rsqrt(v + 1e-5) * lng_ref[...] + lnb_ref[...]
    z = jax.nn.gelu(z)
    z = z + pooled
    out = jnp.dot(z, w2_ref[...], preferred_element_type=jnp.float32)
    out_ref[...] = out + b2_ref[...]


def _head(pooled, params):
    return pl.pallas_call(
        _head_body,
        out_shape=jax.ShapeDtypeStruct((G, 2 * LAT), jnp.float32),
    )(pooled, params['fc1_w'], params['fc1_b'][None, :],
      params['ln_g'][None, :], params['ln_b'][None, :],
      params['fc2_w'], params['fc2_b'][None, :])


def kernel(x, edge_index, batch, params):
    src = edge_index[0]
    dst = edge_index[1]
    h = _bn(x, params['in_bn_g'], params['in_bn_b'])
    hidden = []
    for l in range(L):
        c = params['convs'][l]
        agg = jnp.zeros(h.shape, h.dtype).at[dst].add(h[src])
        z = (1.0 + c['eps']) * h + agg
        z = z @ c['fc1_w'] + c['fc1_b']
        z = _bn(z, c['bn1_g'], c['bn1_b'])
        z = jax.nn.gelu(z)
        z = z @ c['fc2_w'] + c['fc2_b']
        z = _bn(z, params['bns_g'][l], params['bns_b'][l])
        z = jax.nn.gelu(z)
        h = z
        hidden.append(z)
    x_all = jnp.stack(hidden, axis=0)
    scores = jnp.mean(x_all * params['att_w'][:, None, :], axis=-1)
    alpha = jax.nn.softmax(scores, axis=0)
    hjk = jnp.sum(x_all * alpha[..., None], axis=0)
    sums = jax.ops.segment_sum(hjk, batch, num_segments=G)
    counts = jax.ops.segment_sum(jnp.ones((N,), hjk.dtype), batch, num_segments=G)
    mean = sums / jnp.clip(counts, 1.0)[:, None]
    mx = jax.ops.segment_max(hjk, batch, num_segments=G)
    mx = jnp.where(counts[:, None] > 0, mx, 0.0)
    pw = jax.nn.softmax(params['pool_w'])
    pooled = sums * pw[0] + mean * pw[1] + mx * pw[2]
    out = _head(pooled, params)
    mu, logvar = jnp.split(out, 2, axis=-1)
    return (mu, logvar)


# speedup vs baseline: 1.0321x; 1.0320x over previous
import functools

import jax
import jax.numpy as jnp
from jax.experimental import pallas as pl
from jax.experimental.pallas import tpu as pltpu

N = 10000
E = 320000
D = 128
H = 256
LAT = 64
L = 3
G = 64

_BN_SCALE = 1.0 / (1.0 + 1e-5) ** 0.5
_BLK = 1000
_GRID = N // _BLK


def _full(shape):
    return pl.BlockSpec(shape, lambda i: tuple(0 for _ in shape))


def _rows(width):
    return pl.BlockSpec((_BLK, width), lambda i: (i, 0))


def _bn_in_body(x_ref, c_ref, b_ref, o_ref):
    o_ref[...] = x_ref[...] * c_ref[...] + b_ref[...]


def _bn_in(x, c, b):
    return pl.pallas_call(
        _bn_in_body,
        grid=(_GRID,),
        in_specs=[_rows(D), _full((1, D)), _full((1, D))],
        out_specs=_rows(D),
        out_shape=jax.ShapeDtypeStruct((N, D), jnp.float32),
    )(x, c[None, :], b[None, :])


def _mlp_body(h_ref, agg_ref, eps_ref, w1_ref, b1_ref, w2_ref, b2_ref,
              lo_ref, hi_ref):
    u = (1.0 + eps_ref[0]) * h_ref[...] + agg_ref[...]
    z = jnp.dot(u, w1_ref[...], preferred_element_type=jnp.float32)
    z = jax.nn.gelu(z + b1_ref[...])
    z = jnp.dot(z, w2_ref[...], preferred_element_type=jnp.float32)
    z = jax.nn.gelu(z + b2_ref[...])
    lo_ref[...] = z[:, :128]
    hi_ref[...] = z[:, 128:]


def _mlp(h, agg, eps, w1, b1, w2, b2):
    hin = h.shape[1]
    return pl.pallas_call(
        _mlp_body,
        grid=(_GRID,),
        in_specs=[_rows(hin), _rows(hin), _full((1,)), _full((hin, H)),
                  _full((1, H)), _full((H, H)), _full((1, H))],
        out_specs=[_rows(128), _rows(128)],
        out_shape=[jax.ShapeDtypeStruct((N, 128), jnp.float32),
                   jax.ShapeDtypeStruct((N, 128), jnp.float32)],
    )(h, agg, eps[None], w1, b1[None, :], w2, b2[None, :])


def _jk_body(h1l, h1h, h2l, h2h, h3l, h3h, batch_ref, attT_ref,
             lo_ref, hi_ref, sums_ref, cnts_ref):
    i = pl.program_id(0)

    @pl.when(i == 0)
    def _init():
        sums_ref[...] = jnp.zeros_like(sums_ref)
        cnts_ref[...] = jnp.zeros_like(cnts_ref)

    h1 = jnp.concatenate([h1l[...], h1h[...]], axis=1)
    h2 = jnp.concatenate([h2l[...], h2h[...]], axis=1)
    h3 = jnp.concatenate([h3l[...], h3h[...]], axis=1)
    attT = attT_ref[...]
    s1 = jnp.dot(h1, attT[:, 0:1], preferred_element_type=jnp.float32)
    s2 = jnp.dot(h2, attT[:, 1:2], preferred_element_type=jnp.float32)
    s3 = jnp.dot(h3, attT[:, 2:3], preferred_element_type=jnp.float32)
    m = jnp.maximum(jnp.maximum(s1, s2), s3)
    e1 = jnp.exp(s1 - m)
    e2 = jnp.exp(s2 - m)
    e3 = jnp.exp(s3 - m)
    inv = 1.0 / (e1 + e2 + e3)
    hjk = (h1 * e1 + h2 * e2 + h3 * e3) * inv
    lo_ref[...] = hjk[:, :128]
    hi_ref[...] = hjk[:, 128:]

    b = batch_ref[0, 0, :]
    gids = jax.lax.broadcasted_iota(jnp.int32, (_BLK, G), 1)
    maskf = (b[:, None] == gids).astype(jnp.float32)
    dn = (((0,), (0,)), ((), ()))
    sums_ref[...] += jax.lax.dot_general(maskf, hjk, dn,
                                         preferred_element_type=jnp.float32)
    cnts_ref[...] += jax.lax.dot_general(
        maskf, jnp.ones((_BLK, 128), jnp.float32), dn,
        preferred_element_type=jnp.float32)


def _jk(parts, batch3, attT):
    (h1l, h1h), (h2l, h2h), (h3l, h3h) = parts
    return pl.pallas_call(
        _jk_body,
        grid=(_GRID,),
        in_specs=[_rows(128)] * 6 + [
            pl.BlockSpec((1, 1, _BLK), lambda i: (i, 0, 0)),
            _full((H, 3))],
        out_specs=[_rows(128), _rows(128), _full((G, H)), _full((G, 128))],
        out_shape=[jax.ShapeDtypeStruct((N, 128), jnp.float32),
                   jax.ShapeDtypeStruct((N, 128), jnp.float32),
                   jax.ShapeDtypeStruct((G, H), jnp.float32),
                   jax.ShapeDtypeStruct((G, 128), jnp.float32)],
    )(h1l, h1h, h2l, h2h, h3l, h3h, batch3, attT)


def _head_body(sums_ref, cnts_ref, mx_ref, poolw_ref, w1_ref, b1_ref,
               lng_ref, lnb_ref, w2_ref, b2_ref, out_ref):
    sums = sums_ref[...]
    cnt = cnts_ref[:, 0:1]
    mean = sums / jnp.maximum(cnt, 1.0)
    mx = jnp.where(cnt > 0.0, mx_ref[...], 0.0)
    pw = jax.nn.softmax(poolw_ref[...], axis=1)
    pooled = sums * pw[:, 0:1] + mean * pw[:, 1:2] + mx * pw[:, 2:3]
    z = jnp.dot(pooled, w1_ref[...], preferred_element_type=jnp.float32)
    z = z + b1_ref[...]
    mu_ = jnp.mean(z, axis=-1, keepdims=True)
    var_ = jnp.mean((z - mu_) ** 2, axis=-1, keepdims=True)
    z = (z - mu_) * jax.lax.rsqrt(var_ + 1e-5) * lng_ref[...] + lnb_ref[...]
    z = jax.nn.gelu(z) + pooled
    out = jnp.dot(z, w2_ref[...], preferred_element_type=jnp.float32)
    out_ref[...] = out + b2_ref[...]


def _head(sums, cnts, mx, pool_w, p):
    return pl.pallas_call(
        _head_body,
        out_shape=jax.ShapeDtypeStruct((G, 2 * LAT), jnp.float32),
    )(sums, cnts, mx, pool_w, p['fc1_w'], p['fc1_b'][None, :],
      p['ln_g'][None, :], p['ln_b'][None, :], p['fc2_w'], p['fc2_b'][None, :])


def kernel(x, edge_index, batch, params):
    src = edge_index[0]
    dst = edge_index[1]
    p = params

    h0 = _bn_in(x, _BN_SCALE * p['in_bn_g'], p['in_bn_b'])

    hs = []
    h = h0
    for l in range(L):
        c = p['convs'][l]
        s1 = _BN_SCALE * c['bn1_g']
        w1 = c['fc1_w'] * s1[None, :]
        b1 = c['fc1_b'] * s1 + c['bn1_b']
        s2 = _BN_SCALE * p['bns_g'][l]
        w2 = c['fc2_w'] * s2[None, :]
        b2 = c['fc2_b'] * s2 + p['bns_b'][l]
        agg = jnp.zeros(h.shape, h.dtype).at[dst].add(h[src])
        parts = _mlp(h, agg, c['eps'], w1, b1, w2, b2)
        hs.append(parts)
        h = jnp.concatenate(parts, axis=1)

    batch3 = batch.reshape(_GRID, 1, _BLK)
    attT = (p['att_w'] / H).T
    hjk_lo, hjk_hi, sums, cnts = _jk(hs, batch3, attT)

    hjk = jnp.concatenate([hjk_lo, hjk_hi], axis=1)
    mx = jax.ops.segment_max(hjk, batch, num_segments=G)
    mx = jnp.maximum(mx, -1e30)

    out = _head(sums, cnts, mx, p['pool_w'][None, :], p)
    mu, logvar = jnp.split(out, 2, axis=-1)
    return (mu, logvar)


# speedup vs baseline: 4.1697x; 4.0402x over previous
import functools

import jax
import jax.numpy as jnp
from jax import lax
from jax.experimental import pallas as pl
from jax.experimental.pallas import tpu as pltpu
from jax.experimental.pallas import tpu_sc as plsc

N = 10000
E = 320000
D = 128
H = 256
LAT = 64
L = 3
G = 64

_BN_SCALE = 1.0 / (1.0 + 1e-5) ** 0.5
_BLK = 1000
_GRID = N // _BLK


def _full(shape):
    return pl.BlockSpec(shape, lambda i: tuple(0 for _ in shape))


def _rows(width):
    return pl.BlockSpec((_BLK, width), lambda i: (i, 0))


_NTILES = 16
_CHUNK = 128
_ZROWS = 200
_NCHUNKS = N // _ZROWS


def _zero_block(zblk):
    def zrow(r, _):
        for j in range(8):
            zblk[r, pl.ds(j * 16, 16)] = jnp.zeros((16,), jnp.float32)
        return 0
    lax.fori_loop(0, _ZROWS, zrow, 0)


def _row_chunks(s, fn):
    for kk in range((_NCHUNKS + _NTILES - 1) // _NTILES):
        j = s + _NTILES * kk

        @pl.when(j < _NCHUNKS)
        def _():
            fn(j * _ZROWS)


def _zero_acc(zblk, acc, s):
    _zero_block(zblk)
    _row_chunks(s, lambda r0: pltpu.sync_copy(zblk, acc.at[pl.ds(r0, _ZROWS)]))


def _edge_loop(table, src_hbm, dst_hbm, sidx, didx, didx_t, rows, acc, sem,
               base, nfull, tail):
    def step(k, _):
        off = base + k * _CHUNK
        pltpu.sync_copy(src_hbm.at[pl.ds(off, _CHUNK)], sidx)
        pltpu.sync_copy(dst_hbm.at[pl.ds(off, _CHUNK)], didx)
        pltpu.async_copy(table.at[sidx], rows, sem).wait()
        pltpu.sync_copy(rows, acc.at[didx], add=True)
        return 0
    lax.fori_loop(0, nfull, step, 0)
    if tail:
        off = base + nfull * _CHUNK
        pltpu.sync_copy(src_hbm.at[pl.ds(off, tail)], sidx.at[pl.ds(0, tail)])
        pltpu.sync_copy(dst_hbm.at[pl.ds(off, tail)], didx_t)
        pltpu.async_copy(table.at[sidx.at[pl.ds(0, tail)]],
                         rows.at[pl.ds(0, tail)], sem).wait()
        pltpu.sync_copy(rows.at[pl.ds(0, tail)], acc.at[didx_t], add=True)


def _sc_scratch(tail):
    return [
        pltpu.VMEM((_CHUNK,), jnp.int32),
        pltpu.VMEM((_CHUNK,), jnp.int32),
        pltpu.VMEM((max(tail, 8),), jnp.int32),
        pltpu.VMEM((_CHUNK, 128), jnp.float32),
        pltpu.VMEM((_ZROWS, 128), jnp.float32),
        pltpu.VMEM_SHARED((N, 128), jnp.float32),
        pltpu.SemaphoreType.DMA,
    ]
_MESH = plsc.VectorSubcoreMesh(core_axis_name="c", subcore_axis_name="s")


def _agg_half(h_lo, h_hi, src, dst):
    nfull, tail = (E // _NTILES) // _CHUNK, (E // _NTILES) % _CHUNK

    @functools.partial(
        pl.kernel,
        out_type=jax.ShapeDtypeStruct((N, H), jnp.float32),
        mesh=_MESH,
        scratch_types=_sc_scratch(tail),
    )
    def k(hlo_hbm, hhi_hbm, src_hbm, dst_hbm, out_hbm,
          sidx, didx, didx_t, rows, zblk, acc, sem):
        c = lax.axis_index("c")
        s = lax.axis_index("s")
        _zero_acc(zblk, acc, s)
        plsc.subcore_barrier()
        base = s * (E // _NTILES)
        args = (src_hbm, dst_hbm, sidx, didx, didx_t, rows, acc, sem,
                base, nfull, tail)

        @pl.when(c == 0)
        def _():
            _edge_loop(hlo_hbm, *args)

        @pl.when(c == 1)
        def _():
            _edge_loop(hhi_hbm, *args)

        plsc.subcore_barrier()

        @pl.when(c == 0)
        def _():
            _row_chunks(s, lambda r0: pltpu.sync_copy(
                acc.at[pl.ds(r0, _ZROWS)],
                out_hbm.at[pl.ds(r0, _ZROWS), pl.ds(0, 128)]))

        @pl.when(c == 1)
        def _():
            _row_chunks(s, lambda r0: pltpu.sync_copy(
                acc.at[pl.ds(r0, _ZROWS)],
                out_hbm.at[pl.ds(r0, _ZROWS), pl.ds(128, 128)]))

    return k(h_lo, h_hi, src, dst)


def _agg_first(h0, src, dst):
    per_tile = E // (2 * _NTILES)
    nfull, tail = per_tile // _CHUNK, per_tile % _CHUNK

    @functools.partial(
        pl.kernel,
        out_type=jax.ShapeDtypeStruct((2, N, 128), jnp.float32),
        mesh=_MESH,
        scratch_types=_sc_scratch(tail),
    )
    def k(h_hbm, src_hbm, dst_hbm, out_hbm,
          sidx, didx, didx_t, rows, zblk, acc, sem):
        c = lax.axis_index("c")
        s = lax.axis_index("s")
        _zero_acc(zblk, acc, s)
        plsc.subcore_barrier()
        base = (c * _NTILES + s) * per_tile
        _edge_loop(h_hbm, src_hbm, dst_hbm, sidx, didx, didx_t, rows, acc,
                   sem, base, nfull, tail)
        plsc.subcore_barrier()
        _row_chunks(s, lambda r0: pltpu.sync_copy(
            acc.at[pl.ds(r0, _ZROWS)], out_hbm.at[c, pl.ds(r0, _ZROWS)]))

    return k(h0, src, dst)


def _bn_in_body(x_ref, c_ref, b_ref, o_ref):
    o_ref[...] = x_ref[...] * c_ref[...] + b_ref[...]


def _bn_in(x, c, b):
    return pl.pallas_call(
        _bn_in_body,
        grid=(_GRID,),
        in_specs=[_rows(D), _full((1, D)), _full((1, D))],
        out_specs=_rows(D),
        out_shape=jax.ShapeDtypeStruct((N, D), jnp.float32),
    )(x, c[None, :], b[None, :])


def _mlp_body(h_ref, agg_ref, eps_ref, w1_ref, b1_ref, w2_ref, b2_ref,
              lo_ref, hi_ref):
    if agg_ref.shape[0] == 2:
        agg = agg_ref[0] + agg_ref[1]
    else:
        agg = agg_ref[...]
    u = (1.0 + eps_ref[0]) * h_ref[...] + agg
    z = jnp.dot(u, w1_ref[...], preferred_element_type=jnp.float32)
    z = jax.nn.gelu(z + b1_ref[...])
    z = jnp.dot(z, w2_ref[...], preferred_element_type=jnp.float32)
    z = jax.nn.gelu(z + b2_ref[...])
    lo_ref[...] = z[:, :128]
    hi_ref[...] = z[:, 128:]


def _mlp(h, agg, eps, w1, b1, w2, b2):
    hin = h.shape[1]
    if agg.ndim == 3:
        agg_spec = pl.BlockSpec((2, _BLK, 128), lambda i: (0, i, 0))
    else:
        agg_spec = _rows(hin)
    return pl.pallas_call(
        _mlp_body,
        grid=(_GRID,),
        in_specs=[_rows(hin), agg_spec, _full((1,)), _full((hin, H)),
                  _full((1, H)), _full((H, H)), _full((1, H))],
        out_specs=[_rows(128), _rows(128)],
        out_shape=[jax.ShapeDtypeStruct((N, 128), jnp.float32),
                   jax.ShapeDtypeStruct((N, 128), jnp.float32)],
    )(h, agg, eps[None], w1, b1[None, :], w2, b2[None, :])


def _jk_body(h1l, h1h, h2l, h2h, h3l, h3h, batch_ref, attT_ref,
             lo_ref, hi_ref, sums_ref, cnts_ref):
    i = pl.program_id(0)

    @pl.when(i == 0)
    def _init():
        sums_ref[...] = jnp.zeros_like(sums_ref)
        cnts_ref[...] = jnp.zeros_like(cnts_ref)

    h1 = jnp.concatenate([h1l[...], h1h[...]], axis=1)
    h2 = jnp.concatenate([h2l[...], h2h[...]], axis=1)
    h3 = jnp.concatenate([h3l[...], h3h[...]], axis=1)
    attT = attT_ref[...]
    s1 = jnp.dot(h1, attT[:, 0:1], preferred_element_type=jnp.float32)
    s2 = jnp.dot(h2, attT[:, 1:2], preferred_element_type=jnp.float32)
    s3 = jnp.dot(h3, attT[:, 2:3], preferred_element_type=jnp.float32)
    m = jnp.maximum(jnp.maximum(s1, s2), s3)
    e1 = jnp.exp(s1 - m)
    e2 = jnp.exp(s2 - m)
    e3 = jnp.exp(s3 - m)
    inv = 1.0 / (e1 + e2 + e3)
    hjk = (h1 * e1 + h2 * e2 + h3 * e3) * inv
    lo_ref[...] = hjk[:, :128]
    hi_ref[...] = hjk[:, 128:]

    b = batch_ref[0, 0, :]
    gids = jax.lax.broadcasted_iota(jnp.int32, (_BLK, G), 1)
    maskf = (b[:, None] == gids).astype(jnp.float32)
    dn = (((0,), (0,)), ((), ()))
    sums_ref[...] += jax.lax.dot_general(maskf, hjk, dn,
                                         preferred_element_type=jnp.float32)
    cnts_ref[...] += jax.lax.dot_general(
        maskf, jnp.ones((_BLK, 128), jnp.float32), dn,
        preferred_element_type=jnp.float32)


def _jk(parts, batch3, attT):
    (h1l, h1h), (h2l, h2h), (h3l, h3h) = parts
    return pl.pallas_call(
        _jk_body,
        grid=(_GRID,),
        in_specs=[_rows(128)] * 6 + [
            pl.BlockSpec((1, 1, _BLK), lambda i: (i, 0, 0)),
            _full((H, 3))],
        out_specs=[_rows(128), _rows(128), _full((G, H)), _full((G, 128))],
        out_shape=[jax.ShapeDtypeStruct((N, 128), jnp.float32),
                   jax.ShapeDtypeStruct((N, 128), jnp.float32),
                   jax.ShapeDtypeStruct((G, H), jnp.float32),
                   jax.ShapeDtypeStruct((G, 128), jnp.float32)],
    )(h1l, h1h, h2l, h2h, h3l, h3h, batch3, attT)


def _head_body(sums_ref, cnts_ref, mx_ref, poolw_ref, w1_ref, b1_ref,
               lng_ref, lnb_ref, w2_ref, b2_ref, out_ref):
    sums = sums_ref[...]
    cnt = cnts_ref[:, 0:1]
    mean = sums / jnp.maximum(cnt, 1.0)
    mx = jnp.where(cnt > 0.0, mx_ref[...], 0.0)
    pw = jax.nn.softmax(poolw_ref[...], axis=1)
    pooled = sums * pw[:, 0:1] + mean * pw[:, 1:2] + mx * pw[:, 2:3]
    z = jnp.dot(pooled, w1_ref[...], preferred_element_type=jnp.float32)
    z = z + b1_ref[...]
    mu_ = jnp.mean(z, axis=-1, keepdims=True)
    var_ = jnp.mean((z - mu_) ** 2, axis=-1, keepdims=True)
    z = (z - mu_) * jax.lax.rsqrt(var_ + 1e-5) * lng_ref[...] + lnb_ref[...]
    z = jax.nn.gelu(z) + pooled
    out = jnp.dot(z, w2_ref[...], preferred_element_type=jnp.float32)
    out_ref[...] = out + b2_ref[...]


def _head(sums, cnts, mx, pool_w, p):
    return pl.pallas_call(
        _head_body,
        out_shape=jax.ShapeDtypeStruct((G, 2 * LAT), jnp.float32),
    )(sums, cnts, mx, pool_w, p['fc1_w'], p['fc1_b'][None, :],
      p['ln_g'][None, :], p['ln_b'][None, :], p['fc2_w'], p['fc2_b'][None, :])


def kernel(x, edge_index, batch, params):
    src = edge_index[0]
    dst = edge_index[1]
    p = params

    h0 = _bn_in(x, _BN_SCALE * p['in_bn_g'], p['in_bn_b'])

    hs = []
    h = h0
    for l in range(L):
        c = p['convs'][l]
        s1 = _BN_SCALE * c['bn1_g']
        w1 = c['fc1_w'] * s1[None, :]
        b1 = c['fc1_b'] * s1 + c['bn1_b']
        s2 = _BN_SCALE * p['bns_g'][l]
        w2 = c['fc2_w'] * s2[None, :]
        b2 = c['fc2_b'] * s2 + p['bns_b'][l]
        if l == 0:
            agg = _agg_first(h, src, dst)
        else:
            agg = _agg_half(hs[-1][0], hs[-1][1], src, dst)
        parts = _mlp(h, agg, c['eps'], w1, b1, w2, b2)
        hs.append(parts)
        h = jnp.concatenate(parts, axis=1)

    batch3 = batch.reshape(_GRID, 1, _BLK)
    attT = (p['att_w'] / H).T
    hjk_lo, hjk_hi, sums, cnts = _jk(hs, batch3, attT)

    hjk = jnp.concatenate([hjk_lo, hjk_hi], axis=1)
    mx = jax.ops.segment_max(hjk, batch, num_segments=G)
    mx = jnp.maximum(mx, -1e30)

    out = _head(sums, cnts, mx, p['pool_w'][None, :], p)
    mu, logvar = jnp.split(out, 2, axis=-1)
    return (mu, logvar)


# speedup vs baseline: 4.3998x; 1.0552x over previous
import functools

import jax
import jax.numpy as jnp
from jax import lax
from jax.experimental import pallas as pl
from jax.experimental.pallas import tpu as pltpu
from jax.experimental.pallas import tpu_sc as plsc

N = 10000
E = 320000
D = 128
H = 256
LAT = 64
L = 3
G = 64

_BN_SCALE = 1.0 / (1.0 + 1e-5) ** 0.5
_BLK = 1000
_GRID = N // _BLK


def _full(shape):
    return pl.BlockSpec(shape, lambda i: tuple(0 for _ in shape))


def _rows(width):
    return pl.BlockSpec((_BLK, width), lambda i: (i, 0))


_NTILES = 16
_CHUNK = 128
_ZROWS = 200
_NCHUNKS = N // _ZROWS


def _zero_block(zblk):
    def zrow(r, _):
        for j in range(8):
            zblk[r, pl.ds(j * 16, 16)] = jnp.zeros((16,), jnp.float32)
        return 0
    lax.fori_loop(0, _ZROWS, zrow, 0)


def _row_chunks(s, fn):
    for kk in range((_NCHUNKS + _NTILES - 1) // _NTILES):
        j = s + _NTILES * kk

        @pl.when(j < _NCHUNKS)
        def _():
            fn(j * _ZROWS)


def _zero_acc(zblk, acc, s):
    _zero_block(zblk)
    _row_chunks(s, lambda r0: pltpu.sync_copy(zblk, acc.at[pl.ds(r0, _ZROWS)]))


def _edge_loop(table, src_hbm, dst_hbm, sidx, didx, didx_t, rows, acc, sem,
               base, nfull, tail):
    def step(k, _):
        off = base + k * _CHUNK
        pltpu.sync_copy(src_hbm.at[pl.ds(off, _CHUNK)], sidx)
        pltpu.sync_copy(dst_hbm.at[pl.ds(off, _CHUNK)], didx)
        pltpu.async_copy(table.at[sidx], rows, sem).wait()
        pltpu.sync_copy(rows, acc.at[didx], add=True)
        return 0
    lax.fori_loop(0, nfull, step, 0)
    if tail:
        off = base + nfull * _CHUNK
        pltpu.sync_copy(src_hbm.at[pl.ds(off, tail)], sidx.at[pl.ds(0, tail)])
        pltpu.sync_copy(dst_hbm.at[pl.ds(off, tail)], didx_t)
        pltpu.async_copy(table.at[sidx.at[pl.ds(0, tail)]],
                         rows.at[pl.ds(0, tail)], sem).wait()
        pltpu.sync_copy(rows.at[pl.ds(0, tail)], acc.at[didx_t], add=True)


def _sc_scratch(tail):
    return [
        pltpu.VMEM((_CHUNK,), jnp.int32),
        pltpu.VMEM((_CHUNK,), jnp.int32),
        pltpu.VMEM((max(tail, 8),), jnp.int32),
        pltpu.VMEM((_CHUNK, 128), jnp.float32),
        pltpu.VMEM((_ZROWS, 128), jnp.float32),
        pltpu.VMEM_SHARED((N, 128), jnp.float32),
        pltpu.SemaphoreType.DMA,
    ]
_MESH = plsc.VectorSubcoreMesh(core_axis_name="c", subcore_axis_name="s")


def _agg_half(h_lo, h_hi, src, dst):
    nfull, tail = (E // _NTILES) // _CHUNK, (E // _NTILES) % _CHUNK

    @functools.partial(
        pl.kernel,
        out_type=jax.ShapeDtypeStruct((N, H), jnp.float32),
        mesh=_MESH,
        scratch_types=_sc_scratch(tail),
    )
    def k(hlo_hbm, hhi_hbm, src_hbm, dst_hbm, out_hbm,
          sidx, didx, didx_t, rows, zblk, acc, sem):
        c = lax.axis_index("c")
        s = lax.axis_index("s")
        _zero_acc(zblk, acc, s)
        plsc.subcore_barrier()
        base = s * (E // _NTILES)
        args = (src_hbm, dst_hbm, sidx, didx, didx_t, rows, acc, sem,
                base, nfull, tail)

        @pl.when(c == 0)
        def _():
            _edge_loop(hlo_hbm, *args)

        @pl.when(c == 1)
        def _():
            _edge_loop(hhi_hbm, *args)

        plsc.subcore_barrier()

        @pl.when(c == 0)
        def _():
            _row_chunks(s, lambda r0: pltpu.sync_copy(
                acc.at[pl.ds(r0, _ZROWS)],
                out_hbm.at[pl.ds(r0, _ZROWS), pl.ds(0, 128)]))

        @pl.when(c == 1)
        def _():
            _row_chunks(s, lambda r0: pltpu.sync_copy(
                acc.at[pl.ds(r0, _ZROWS)],
                out_hbm.at[pl.ds(r0, _ZROWS), pl.ds(128, 128)]))

    return k(h_lo, h_hi, src, dst)


def _agg_first(h0, src, dst):
    per_tile = E // (2 * _NTILES)
    nfull, tail = per_tile // _CHUNK, per_tile % _CHUNK

    @functools.partial(
        pl.kernel,
        out_type=jax.ShapeDtypeStruct((2, N, 128), jnp.float32),
        mesh=_MESH,
        scratch_types=_sc_scratch(tail),
    )
    def k(h_hbm, src_hbm, dst_hbm, out_hbm,
          sidx, didx, didx_t, rows, zblk, acc, sem):
        c = lax.axis_index("c")
        s = lax.axis_index("s")
        _zero_acc(zblk, acc, s)
        plsc.subcore_barrier()
        base = (c * _NTILES + s) * per_tile
        _edge_loop(h_hbm, src_hbm, dst_hbm, sidx, didx, didx_t, rows, acc,
                   sem, base, nfull, tail)
        plsc.subcore_barrier()
        _row_chunks(s, lambda r0: pltpu.sync_copy(
            acc.at[pl.ds(r0, _ZROWS)], out_hbm.at[c, pl.ds(r0, _ZROWS)]))

    return k(h0, src, dst)


_MROWS = 80
_MCHUNKS = N // _MROWS


def _segmax(hjk_lo, hjk_hi, batch):
    nw = 2 * _NTILES

    @functools.partial(
        pl.kernel,
        out_type=jax.ShapeDtypeStruct((nw, G, H), jnp.float32),
        mesh=_MESH,
        scratch_types=[
            pltpu.VMEM((_MROWS, 128), jnp.float32),
            pltpu.VMEM((_MROWS, 128), jnp.float32),
            pltpu.VMEM((_MROWS,), jnp.int32),
            pltpu.VMEM((G, H), jnp.float32),
        ],
    )
    def k(lo_hbm, hi_hbm, batch_hbm, out_hbm, rlo, rhi, bat, maxs):
        c = lax.axis_index("c")
        s = lax.axis_index("s")
        wid = c * _NTILES + s

        def mrow(r, _):
            for j in range(16):
                maxs[r, pl.ds(j * 16, 16)] = jnp.full((16,), -1e30, jnp.float32)
            return 0
        lax.fori_loop(0, G, mrow, 0)

        def do_chunk(cid):
            r0 = cid * _MROWS
            pltpu.sync_copy(lo_hbm.at[pl.ds(r0, _MROWS)], rlo)
            pltpu.sync_copy(hi_hbm.at[pl.ds(r0, _MROWS)], rhi)
            pltpu.sync_copy(batch_hbm.at[pl.ds(r0, _MROWS)], bat)

            def rowgrp(q, _):
                bv = bat[pl.ds(q * 16, 16)]
                for r in range(16):
                    g = bv[r]
                    rr = q * 16 + r
                    for j in range(8):
                        sl = pl.ds(j * 16, 16)
                        sh = pl.ds(j * 16 + 128, 16)
                        maxs[g, sl] = jnp.maximum(maxs[g, sl], rlo[rr, sl])
                        maxs[g, sh] = jnp.maximum(maxs[g, sh], rhi[rr, sl])
                return 0
            lax.fori_loop(0, _MROWS // 16, rowgrp, 0)

        for kk in range((_MCHUNKS + nw - 1) // nw):
            cid = wid + nw * kk

            @pl.when(cid < _MCHUNKS)
            def _():
                do_chunk(cid)

        pltpu.sync_copy(maxs, out_hbm.at[wid])

    return k(hjk_lo, hjk_hi, batch)


def _bn_in_body(x_ref, c_ref, b_ref, o_ref):
    o_ref[...] = x_ref[...] * c_ref[...] + b_ref[...]


def _bn_in(x, c, b):
    return pl.pallas_call(
        _bn_in_body,
        grid=(_GRID,),
        in_specs=[_rows(D), _full((1, D)), _full((1, D))],
        out_specs=_rows(D),
        out_shape=jax.ShapeDtypeStruct((N, D), jnp.float32),
    )(x, c[None, :], b[None, :])


def _mlp_body(h_ref, agg_ref, eps_ref, w1_ref, b1_ref, w2_ref, b2_ref,
              lo_ref, hi_ref):
    if agg_ref.shape[0] == 2:
        agg = agg_ref[0] + agg_ref[1]
    else:
        agg = agg_ref[...]
    u = (1.0 + eps_ref[0]) * h_ref[...] + agg
    z = jnp.dot(u, w1_ref[...], preferred_element_type=jnp.float32)
    z = jax.nn.gelu(z + b1_ref[...])
    z = jnp.dot(z, w2_ref[...], preferred_element_type=jnp.float32)
    z = jax.nn.gelu(z + b2_ref[...])
    lo_ref[...] = z[:, :128]
    hi_ref[...] = z[:, 128:]


def _mlp(h, agg, eps, w1, b1, w2, b2):
    hin = h.shape[1]
    if agg.ndim == 3:
        agg_spec = pl.BlockSpec((2, _BLK, 128), lambda i: (0, i, 0))
    else:
        agg_spec = _rows(hin)
    return pl.pallas_call(
        _mlp_body,
        grid=(_GRID,),
        in_specs=[_rows(hin), agg_spec, _full((1,)), _full((hin, H)),
                  _full((1, H)), _full((H, H)), _full((1, H))],
        out_specs=[_rows(128), _rows(128)],
        out_shape=[jax.ShapeDtypeStruct((N, 128), jnp.float32),
                   jax.ShapeDtypeStruct((N, 128), jnp.float32)],
    )(h, agg, eps[None], w1, b1[None, :], w2, b2[None, :])


def _jk_body(h1l, h1h, h2l, h2h, h3l, h3h, batch_ref, attT_ref,
             lo_ref, hi_ref, sums_ref, cnts_ref):
    i = pl.program_id(0)

    @pl.when(i == 0)
    def _init():
        sums_ref[...] = jnp.zeros_like(sums_ref)
        cnts_ref[...] = jnp.zeros_like(cnts_ref)

    h1 = jnp.concatenate([h1l[...], h1h[...]], axis=1)
    h2 = jnp.concatenate([h2l[...], h2h[...]], axis=1)
    h3 = jnp.concatenate([h3l[...], h3h[...]], axis=1)
    attT = attT_ref[...]
    s1 = jnp.dot(h1, attT[:, 0:1], preferred_element_type=jnp.float32)
    s2 = jnp.dot(h2, attT[:, 1:2], preferred_element_type=jnp.float32)
    s3 = jnp.dot(h3, attT[:, 2:3], preferred_element_type=jnp.float32)
    m = jnp.maximum(jnp.maximum(s1, s2), s3)
    e1 = jnp.exp(s1 - m)
    e2 = jnp.exp(s2 - m)
    e3 = jnp.exp(s3 - m)
    inv = 1.0 / (e1 + e2 + e3)
    hjk = (h1 * e1 + h2 * e2 + h3 * e3) * inv
    lo_ref[...] = hjk[:, :128]
    hi_ref[...] = hjk[:, 128:]

    b = batch_ref[0, 0, :]
    gids = jax.lax.broadcasted_iota(jnp.int32, (_BLK, G), 1)
    maskf = (b[:, None] == gids).astype(jnp.float32)
    dn = (((0,), (0,)), ((), ()))
    sums_ref[...] += jax.lax.dot_general(maskf, hjk, dn,
                                         preferred_element_type=jnp.float32)
    cnts_ref[...] += jax.lax.dot_general(
        maskf, jnp.ones((_BLK, 128), jnp.float32), dn,
        preferred_element_type=jnp.float32)


def _jk(parts, batch3, attT):
    (h1l, h1h), (h2l, h2h), (h3l, h3h) = parts
    return pl.pallas_call(
        _jk_body,
        grid=(_GRID,),
        in_specs=[_rows(128)] * 6 + [
            pl.BlockSpec((1, 1, _BLK), lambda i: (i, 0, 0)),
            _full((H, 3))],
        out_specs=[_rows(128), _rows(128), _full((G, H)), _full((G, 128))],
        out_shape=[jax.ShapeDtypeStruct((N, 128), jnp.float32),
                   jax.ShapeDtypeStruct((N, 128), jnp.float32),
                   jax.ShapeDtypeStruct((G, H), jnp.float32),
                   jax.ShapeDtypeStruct((G, 128), jnp.float32)],
    )(h1l, h1h, h2l, h2h, h3l, h3h, batch3, attT)


def _head_body(sums_ref, cnts_ref, mxp_ref, poolw_ref, w1_ref, b1_ref,
               lng_ref, lnb_ref, w2_ref, b2_ref, out_ref):
    sums = sums_ref[...]
    cnt = cnts_ref[:, 0:1]
    mean = sums / jnp.maximum(cnt, 1.0)
    mx = mxp_ref[pl.ds(0, G), :]
    for j in range(1, 2 * _NTILES):
        mx = jnp.maximum(mx, mxp_ref[pl.ds(j * G, G), :])
    mx = jnp.where(cnt > 0.0, mx, 0.0)
    pw = jax.nn.softmax(poolw_ref[...], axis=1)
    pooled = sums * pw[:, 0:1] + mean * pw[:, 1:2] + mx * pw[:, 2:3]
    z = jnp.dot(pooled, w1_ref[...], preferred_element_type=jnp.float32)
    z = z + b1_ref[...]
    mu_ = jnp.mean(z, axis=-1, keepdims=True)
    var_ = jnp.mean((z - mu_) ** 2, axis=-1, keepdims=True)
    z = (z - mu_) * jax.lax.rsqrt(var_ + 1e-5) * lng_ref[...] + lnb_ref[...]
    z = jax.nn.gelu(z) + pooled
    out = jnp.dot(z, w2_ref[...], preferred_element_type=jnp.float32)
    out_ref[...] = out + b2_ref[...]


def _head(sums, cnts, mxp, pool_w, p):
    return pl.pallas_call(
        _head_body,
        out_shape=jax.ShapeDtypeStruct((G, 2 * LAT), jnp.float32),
    )(sums, cnts, mxp, pool_w, p['fc1_w'], p['fc1_b'][None, :],
      p['ln_g'][None, :], p['ln_b'][None, :], p['fc2_w'], p['fc2_b'][None, :])


def kernel(x, edge_index, batch, params):
    src = edge_index[0]
    dst = edge_index[1]
    p = params

    h0 = _bn_in(x, _BN_SCALE * p['in_bn_g'], p['in_bn_b'])

    hs = []
    h = h0
    for l in range(L):
        c = p['convs'][l]
        s1 = _BN_SCALE * c['bn1_g']
        w1 = c['fc1_w'] * s1[None, :]
        b1 = c['fc1_b'] * s1 + c['bn1_b']
        s2 = _BN_SCALE * p['bns_g'][l]
        w2 = c['fc2_w'] * s2[None, :]
        b2 = c['fc2_b'] * s2 + p['bns_b'][l]
        if l == 0:
            agg = _agg_first(h, src, dst)
        else:
            agg = _agg_half(hs[-1][0], hs[-1][1], src, dst)
        parts = _mlp(h, agg, c['eps'], w1, b1, w2, b2)
        hs.append(parts)
        h = jnp.concatenate(parts, axis=1)

    batch3 = batch.reshape(_GRID, 1, _BLK)
    attT = (p['att_w'] / H).T
    hjk_lo, hjk_hi, sums, cnts = _jk(hs, batch3, attT)

    mxp = _segmax(hjk_lo, hjk_hi, batch).reshape(2 * _NTILES * G, H)
    out = _head(sums, cnts, mxp, p['pool_w'][None, :], p)
    mu, logvar = jnp.split(out, 2, axis=-1)
    return (mu, logvar)


# speedup vs baseline: 6.8295x; 1.5522x over previous
import functools

import jax
import jax.numpy as jnp
from jax import lax
from jax.experimental import pallas as pl
from jax.experimental.pallas import tpu as pltpu
from jax.experimental.pallas import tpu_sc as plsc

N = 10000
E = 320000
D = 128
H = 256
LAT = 64
L = 3
G = 64

_BN_SCALE = 1.0 / (1.0 + 1e-5) ** 0.5
_BLK = 1000
_GRID = N // _BLK


def _full(shape):
    return pl.BlockSpec(shape, lambda i: tuple(0 for _ in shape))


def _rows(width):
    return pl.BlockSpec((_BLK, width), lambda i: (i, 0))


_NTILES = 16
_CHUNK = 128
_ZROWS = 200
_NCHUNKS = N // _ZROWS


_ZB = 40


def _zero_block(zblk):
    def zrow(r, _):
        for j in range(8):
            zblk[r, pl.ds(j * 16, 16)] = jnp.zeros((16,), jnp.float32)
        return 0
    lax.fori_loop(0, _ZB, zrow, 0)


def _row_chunks(s, fn):
    for kk in range((_NCHUNKS + _NTILES - 1) // _NTILES):
        j = s + _NTILES * kk

        @pl.when(j < _NCHUNKS)
        def _():
            fn(j * _ZROWS)


def _zero_acc(zblk, acc, s):
    _zero_block(zblk)

    def zchunk(r0):
        for t in range(_ZROWS // _ZB):
            pltpu.sync_copy(zblk, acc.at[pl.ds(r0 + t * _ZB, _ZB)])
    _row_chunks(s, zchunk)


def _edge_loop(table, src_hbm, dst_hbm, sA, dA, sB, dB, didx_t,
               rowsA, rowsB, acc, semA, semB, base, nfull, tail):
    assert nfull % 2 == 0

    def load_start(off, sidx, didx, rows, sem):
        pltpu.sync_copy(src_hbm.at[pl.ds(off, _CHUNK)], sidx)
        pltpu.sync_copy(dst_hbm.at[pl.ds(off, _CHUNK)], didx)
        pltpu.async_copy(table.at[sidx], rows, sem)

    load_start(base, sA, dA, rowsA, semA)

    def body(p, _):
        k = 2 * p
        load_start(base + (k + 1) * _CHUNK, sB, dB, rowsB, semB)
        pltpu.make_async_copy(table.at[sA], rowsA, semA).wait()
        pltpu.sync_copy(rowsA, acc.at[dA], add=True)

        @pl.when(k + 2 < nfull)
        def _():
            load_start(base + (k + 2) * _CHUNK, sA, dA, rowsA, semA)

        pltpu.make_async_copy(table.at[sB], rowsB, semB).wait()
        pltpu.sync_copy(rowsB, acc.at[dB], add=True)
        return 0
    lax.fori_loop(0, nfull // 2, body, 0)
    if tail:
        off = base + nfull * _CHUNK
        pltpu.sync_copy(src_hbm.at[pl.ds(off, tail)], sA.at[pl.ds(0, tail)])
        pltpu.sync_copy(dst_hbm.at[pl.ds(off, tail)], didx_t)
        pltpu.async_copy(table.at[sA.at[pl.ds(0, tail)]],
                         rowsA.at[pl.ds(0, tail)], semA).wait()
        pltpu.sync_copy(rowsA.at[pl.ds(0, tail)], acc.at[didx_t], add=True)


def _sc_scratch(tail):
    return [
        pltpu.VMEM((_CHUNK,), jnp.int32),
        pltpu.VMEM((_CHUNK,), jnp.int32),
        pltpu.VMEM((_CHUNK,), jnp.int32),
        pltpu.VMEM((_CHUNK,), jnp.int32),
        pltpu.VMEM((max(tail, 8),), jnp.int32),
        pltpu.VMEM((_CHUNK, 128), jnp.float32),
        pltpu.VMEM((_CHUNK, 128), jnp.float32),
        pltpu.VMEM((_ZB, 128), jnp.float32),
        pltpu.VMEM_SHARED((N, 128), jnp.float32),
        pltpu.SemaphoreType.DMA,
        pltpu.SemaphoreType.DMA,
    ]
_MESH = plsc.VectorSubcoreMesh(core_axis_name="c", subcore_axis_name="s")


def _agg_half(h_lo, h_hi, src, dst):
    nfull, tail = (E // _NTILES) // _CHUNK, (E // _NTILES) % _CHUNK

    @functools.partial(
        pl.kernel,
        out_type=jax.ShapeDtypeStruct((N, H), jnp.float32),
        mesh=_MESH,
        scratch_types=_sc_scratch(tail),
    )
    def k(hlo_hbm, hhi_hbm, src_hbm, dst_hbm, out_hbm,
          sA, dA, sB, dB, didx_t, rowsA, rowsB, zblk, acc, semA, semB):
        c = lax.axis_index("c")
        s = lax.axis_index("s")
        _zero_acc(zblk, acc, s)
        plsc.subcore_barrier()
        base = s * (E // _NTILES)
        args = (src_hbm, dst_hbm, sA, dA, sB, dB, didx_t, rowsA, rowsB,
                acc, semA, semB, base, nfull, tail)

        @pl.when(c == 0)
        def _():
            _edge_loop(hlo_hbm, *args)

        @pl.when(c == 1)
        def _():
            _edge_loop(hhi_hbm, *args)

        plsc.subcore_barrier()

        @pl.when(c == 0)
        def _():
            _row_chunks(s, lambda r0: pltpu.sync_copy(
                acc.at[pl.ds(r0, _ZROWS)],
                out_hbm.at[pl.ds(r0, _ZROWS), pl.ds(0, 128)]))

        @pl.when(c == 1)
        def _():
            _row_chunks(s, lambda r0: pltpu.sync_copy(
                acc.at[pl.ds(r0, _ZROWS)],
                out_hbm.at[pl.ds(r0, _ZROWS), pl.ds(128, 128)]))

    return k(h_lo, h_hi, src, dst)


def _agg_first(h0, src, dst):
    per_tile = E // (2 * _NTILES)
    nfull, tail = per_tile // _CHUNK, per_tile % _CHUNK

    @functools.partial(
        pl.kernel,
        out_type=jax.ShapeDtypeStruct((2, N, 128), jnp.float32),
        mesh=_MESH,
        scratch_types=_sc_scratch(tail),
    )
    def k(h_hbm, src_hbm, dst_hbm, out_hbm,
          sA, dA, sB, dB, didx_t, rowsA, rowsB, zblk, acc, semA, semB):
        c = lax.axis_index("c")
        s = lax.axis_index("s")
        _zero_acc(zblk, acc, s)
        plsc.subcore_barrier()
        base = (c * _NTILES + s) * per_tile
        _edge_loop(h_hbm, src_hbm, dst_hbm, sA, dA, sB, dB, didx_t,
                   rowsA, rowsB, acc, semA, semB, base, nfull, tail)
        plsc.subcore_barrier()
        _row_chunks(s, lambda r0: pltpu.sync_copy(
            acc.at[pl.ds(r0, _ZROWS)], out_hbm.at[c, pl.ds(r0, _ZROWS)]))

    return k(h0, src, dst)


_MROWS = 80
_MCHUNKS = N // _MROWS


def _segmax(hjk_lo, hjk_hi, batch):
    nw = 2 * _NTILES

    @functools.partial(
        pl.kernel,
        out_type=jax.ShapeDtypeStruct((nw, G, H), jnp.float32),
        mesh=_MESH,
        scratch_types=[
            pltpu.VMEM((_MROWS, 128), jnp.float32),
            pltpu.VMEM((_MROWS, 128), jnp.float32),
            pltpu.VMEM((_MROWS,), jnp.int32),
            pltpu.VMEM((G, H), jnp.float32),
        ],
    )
    def k(lo_hbm, hi_hbm, batch_hbm, out_hbm, rlo, rhi, bat, maxs):
        c = lax.axis_index("c")
        s = lax.axis_index("s")
        wid = c * _NTILES + s

        def mrow(r, _):
            for j in range(16):
                maxs[r, pl.ds(j * 16, 16)] = jnp.full((16,), -1e30, jnp.float32)
            return 0
        lax.fori_loop(0, G, mrow, 0)

        def do_chunk(cid):
            r0 = cid * _MROWS
            pltpu.sync_copy(lo_hbm.at[pl.ds(r0, _MROWS)], rlo)
            pltpu.sync_copy(hi_hbm.at[pl.ds(r0, _MROWS)], rhi)
            pltpu.sync_copy(batch_hbm.at[pl.ds(r0, _MROWS)], bat)

            def rowgrp(q, _):
                bv = bat[pl.ds(q * 16, 16)]
                for r in range(16):
                    g = bv[r]
                    rr = q * 16 + r
                    for j in range(8):
                        sl = pl.ds(j * 16, 16)
                        sh = pl.ds(j * 16 + 128, 16)
                        maxs[g, sl] = jnp.maximum(maxs[g, sl], rlo[rr, sl])
                        maxs[g, sh] = jnp.maximum(maxs[g, sh], rhi[rr, sl])
                return 0
            lax.fori_loop(0, _MROWS // 16, rowgrp, 0)

        for kk in range((_MCHUNKS + nw - 1) // nw):
            cid = wid + nw * kk

            @pl.when(cid < _MCHUNKS)
            def _():
                do_chunk(cid)

        pltpu.sync_copy(maxs, out_hbm.at[wid])

    return k(hjk_lo, hjk_hi, batch)


def _bn_in_body(x_ref, c_ref, b_ref, o_ref):
    o_ref[...] = x_ref[...] * c_ref[...] + b_ref[...]


def _bn_in(x, c, b):
    return pl.pallas_call(
        _bn_in_body,
        grid=(_GRID,),
        in_specs=[_rows(D), _full((1, D)), _full((1, D))],
        out_specs=_rows(D),
        out_shape=jax.ShapeDtypeStruct((N, D), jnp.float32),
    )(x, c[None, :], b[None, :])


def _mlp_body(h_ref, agg_ref, eps_ref, w1_ref, b1_ref, w2_ref, b2_ref,
              lo_ref, hi_ref):
    if agg_ref.shape[0] == 2:
        agg = agg_ref[0] + agg_ref[1]
    else:
        agg = agg_ref[...]
    u = (1.0 + eps_ref[0]) * h_ref[...] + agg
    z = jnp.dot(u, w1_ref[...], preferred_element_type=jnp.float32)
    z = jax.nn.gelu(z + b1_ref[...])
    z = jnp.dot(z, w2_ref[...], preferred_element_type=jnp.float32)
    z = jax.nn.gelu(z + b2_ref[...])
    lo_ref[...] = z[:, :128]
    hi_ref[...] = z[:, 128:]


def _mlp(h, agg, eps, w1, b1, w2, b2):
    hin = h.shape[1]
    if agg.ndim == 3:
        agg_spec = pl.BlockSpec((2, _BLK, 128), lambda i: (0, i, 0))
    else:
        agg_spec = _rows(hin)
    return pl.pallas_call(
        _mlp_body,
        grid=(_GRID,),
        in_specs=[_rows(hin), agg_spec, _full((1,)), _full((hin, H)),
                  _full((1, H)), _full((H, H)), _full((1, H))],
        out_specs=[_rows(128), _rows(128)],
        out_shape=[jax.ShapeDtypeStruct((N, 128), jnp.float32),
                   jax.ShapeDtypeStruct((N, 128), jnp.float32)],
    )(h, agg, eps[None], w1, b1[None, :], w2, b2[None, :])


def _jk_body(h1l, h1h, h2l, h2h, h3l, h3h, batch_ref, attT_ref,
             lo_ref, hi_ref, sums_ref, cnts_ref):
    i = pl.program_id(0)

    @pl.when(i == 0)
    def _init():
        sums_ref[...] = jnp.zeros_like(sums_ref)
        cnts_ref[...] = jnp.zeros_like(cnts_ref)

    h1 = jnp.concatenate([h1l[...], h1h[...]], axis=1)
    h2 = jnp.concatenate([h2l[...], h2h[...]], axis=1)
    h3 = jnp.concatenate([h3l[...], h3h[...]], axis=1)
    attT = attT_ref[...]
    s1 = jnp.dot(h1, attT[:, 0:1], preferred_element_type=jnp.float32)
    s2 = jnp.dot(h2, attT[:, 1:2], preferred_element_type=jnp.float32)
    s3 = jnp.dot(h3, attT[:, 2:3], preferred_element_type=jnp.float32)
    m = jnp.maximum(jnp.maximum(s1, s2), s3)
    e1 = jnp.exp(s1 - m)
    e2 = jnp.exp(s2 - m)
    e3 = jnp.exp(s3 - m)
    inv = 1.0 / (e1 + e2 + e3)
    hjk = (h1 * e1 + h2 * e2 + h3 * e3) * inv
    lo_ref[...] = hjk[:, :128]
    hi_ref[...] = hjk[:, 128:]

    b = batch_ref[0, 0, :]
    gids = jax.lax.broadcasted_iota(jnp.int32, (_BLK, G), 1)
    maskf = (b[:, None] == gids).astype(jnp.float32)
    dn = (((0,), (0,)), ((), ()))
    sums_ref[...] += jax.lax.dot_general(maskf, hjk, dn,
                                         preferred_element_type=jnp.float32)
    cnts_ref[...] += jax.lax.dot_general(
        maskf, jnp.ones((_BLK, 128), jnp.float32), dn,
        preferred_element_type=jnp.float32)


def _jk(parts, batch3, attT):
    (h1l, h1h), (h2l, h2h), (h3l, h3h) = parts
    return pl.pallas_call(
        _jk_body,
        grid=(_GRID,),
        in_specs=[_rows(128)] * 6 + [
            pl.BlockSpec((1, 1, _BLK), lambda i: (i, 0, 0)),
            _full((H, 3))],
        out_specs=[_rows(128), _rows(128), _full((G, H)), _full((G, 128))],
        out_shape=[jax.ShapeDtypeStruct((N, 128), jnp.float32),
                   jax.ShapeDtypeStruct((N, 128), jnp.float32),
                   jax.ShapeDtypeStruct((G, H), jnp.float32),
                   jax.ShapeDtypeStruct((G, 128), jnp.float32)],
    )(h1l, h1h, h2l, h2h, h3l, h3h, batch3, attT)


def _head_body(sums_ref, cnts_ref, mxp_ref, poolw_ref, w1_ref, b1_ref,
               lng_ref, lnb_ref, w2_ref, b2_ref, out_ref):
    sums = sums_ref[...]
    cnt = cnts_ref[:, 0:1]
    mean = sums / jnp.maximum(cnt, 1.0)
    mx = mxp_ref[pl.ds(0, G), :]
    for j in range(1, 2 * _NTILES):
        mx = jnp.maximum(mx, mxp_ref[pl.ds(j * G, G), :])
    mx = jnp.where(cnt > 0.0, mx, 0.0)
    pw = jax.nn.softmax(poolw_ref[...], axis=1)
    pooled = sums * pw[:, 0:1] + mean * pw[:, 1:2] + mx * pw[:, 2:3]
    z = jnp.dot(pooled, w1_ref[...], preferred_element_type=jnp.float32)
    z = z + b1_ref[...]
    mu_ = jnp.mean(z, axis=-1, keepdims=True)
    var_ = jnp.mean((z - mu_) ** 2, axis=-1, keepdims=True)
    z = (z - mu_) * jax.lax.rsqrt(var_ + 1e-5) * lng_ref[...] + lnb_ref[...]
    z = jax.nn.gelu(z) + pooled
    out = jnp.dot(z, w2_ref[...], preferred_element_type=jnp.float32)
    out_ref[...] = out + b2_ref[...]


def _head(sums, cnts, mxp, pool_w, p):
    return pl.pallas_call(
        _head_body,
        out_shape=jax.ShapeDtypeStruct((G, 2 * LAT), jnp.float32),
    )(sums, cnts, mxp, pool_w, p['fc1_w'], p['fc1_b'][None, :],
      p['ln_g'][None, :], p['ln_b'][None, :], p['fc2_w'], p['fc2_b'][None, :])


def kernel(x, edge_index, batch, params):
    src = edge_index[0]
    dst = edge_index[1]
    p = params

    h0 = _bn_in(x, _BN_SCALE * p['in_bn_g'], p['in_bn_b'])

    hs = []
    h = h0
    for l in range(L):
        c = p['convs'][l]
        s1 = _BN_SCALE * c['bn1_g']
        w1 = c['fc1_w'] * s1[None, :]
        b1 = c['fc1_b'] * s1 + c['bn1_b']
        s2 = _BN_SCALE * p['bns_g'][l]
        w2 = c['fc2_w'] * s2[None, :]
        b2 = c['fc2_b'] * s2 + p['bns_b'][l]
        if l == 0:
            agg = _agg_first(h, src, dst)
        else:
            agg = _agg_half(hs[-1][0], hs[-1][1], src, dst)
        parts = _mlp(h, agg, c['eps'], w1, b1, w2, b2)
        hs.append(parts)
        h = jnp.concatenate(parts, axis=1)

    batch3 = batch.reshape(_GRID, 1, _BLK)
    attT = (p['att_w'] / H).T
    hjk_lo, hjk_hi, sums, cnts = _jk(hs, batch3, attT)

    mxp = _segmax(hjk_lo, hjk_hi, batch).reshape(2 * _NTILES * G, H)
    out = _head(sums, cnts, mxp, p['pool_w'][None, :], p)
    mu, logvar = jnp.split(out, 2, axis=-1)
    return (mu, logvar)


# speedup vs baseline: 7.3338x; 1.0738x over previous
import functools

import jax
import jax.numpy as jnp
from jax import lax
from jax.experimental import pallas as pl
from jax.experimental.pallas import tpu as pltpu
from jax.experimental.pallas import tpu_sc as plsc

N = 10000
E = 320000
D = 128
H = 256
LAT = 64
L = 3
G = 64

_BN_SCALE = 1.0 / (1.0 + 1e-5) ** 0.5
_BLK = 1000
_GRID = N // _BLK


def _full(shape):
    return pl.BlockSpec(shape, lambda i: tuple(0 for _ in shape))


def _rows(width):
    return pl.BlockSpec((_BLK, width), lambda i: (i, 0))


_NTILES = 16
_CHUNK = 128
_ZROWS = 200
_NCHUNKS = N // _ZROWS


_ZB = 40


def _zero_block(zblk):
    def zrow(r, _):
        for j in range(8):
            zblk[r, pl.ds(j * 16, 16)] = jnp.zeros((16,), jnp.float32)
        return 0
    lax.fori_loop(0, _ZB, zrow, 0)


def _row_chunks(s, fn):
    for kk in range((_NCHUNKS + _NTILES - 1) // _NTILES):
        j = s + _NTILES * kk

        @pl.when(j < _NCHUNKS)
        def _():
            fn(j * _ZROWS)


def _zero_acc(zblk, acc, s):
    _zero_block(zblk)

    def zchunk(r0):
        for t in range(_ZROWS // _ZB):
            pltpu.sync_copy(zblk, acc.at[pl.ds(r0 + t * _ZB, _ZB)])
    _row_chunks(s, zchunk)


_GRP = 8
_EROWS = E // _CHUNK


def _edge_groups(table, src2, dst2, sg, dg, rowsA, rowsB, acc, semA, semB,
                 worker, nworkers, ngroups, rem):
    def do_group(r0):
        pltpu.sync_copy(src2.at[pl.ds(r0, _GRP)], sg)
        pltpu.sync_copy(dst2.at[pl.ds(r0, _GRP)], dg)
        pltpu.async_copy(table.at[sg.at[0]], rowsA, semA)
        for q in range(_GRP // 2):
            pltpu.async_copy(table.at[sg.at[2 * q + 1]], rowsB, semB)
            pltpu.make_async_copy(table.at[sg.at[0]], rowsA, semA).wait()
            pltpu.sync_copy(rowsA, acc.at[dg.at[2 * q]], add=True)
            if q < _GRP // 2 - 1:
                pltpu.async_copy(table.at[sg.at[2 * q + 2]], rowsA, semA)
            pltpu.make_async_copy(table.at[sg.at[0]], rowsB, semB).wait()
            pltpu.sync_copy(rowsB, acc.at[dg.at[2 * q + 1]], add=True)

    def body(gi, _):
        j = worker + nworkers * gi

        @pl.when(j < ngroups)
        def _():
            do_group(pl.multiple_of(j * _GRP, _GRP))
        return 0
    lax.fori_loop(0, (ngroups + nworkers - 1) // nworkers, body, 0)

    if rem:
        @pl.when(worker == 0)
        def _():
            r0 = ngroups * _GRP
            pltpu.sync_copy(src2.at[pl.ds(r0, rem)], sg.at[pl.ds(0, rem)])
            pltpu.sync_copy(dst2.at[pl.ds(r0, rem)], dg.at[pl.ds(0, rem)])
            for q in range(rem):
                pltpu.async_copy(table.at[sg.at[q]], rowsA, semA).wait()
                pltpu.sync_copy(rowsA, acc.at[dg.at[q]], add=True)


def _sc_scratch():
    return [
        pltpu.VMEM((_GRP, _CHUNK), jnp.int32),
        pltpu.VMEM((_GRP, _CHUNK), jnp.int32),
        pltpu.VMEM((_CHUNK, 128), jnp.float32),
        pltpu.VMEM((_CHUNK, 128), jnp.float32),
        pltpu.VMEM((_ZB, 128), jnp.float32),
        pltpu.VMEM_SHARED((N, 128), jnp.float32),
        pltpu.SemaphoreType.DMA,
        pltpu.SemaphoreType.DMA,
    ]
_MESH = plsc.VectorSubcoreMesh(core_axis_name="c", subcore_axis_name="s")


def _agg_half(h_lo, h_hi, src2, dst2):
    ngroups, rem = _EROWS // _GRP, _EROWS % _GRP

    @functools.partial(
        pl.kernel,
        out_type=jax.ShapeDtypeStruct((N, H), jnp.float32),
        mesh=_MESH,
        scratch_types=_sc_scratch(),
    )
    def k(hlo_hbm, hhi_hbm, src_hbm, dst_hbm, out_hbm,
          sg, dg, rowsA, rowsB, zblk, acc, semA, semB):
        c = lax.axis_index("c")
        s = lax.axis_index("s")
        _zero_acc(zblk, acc, s)
        plsc.subcore_barrier()
        args = (src_hbm, dst_hbm, sg, dg, rowsA, rowsB, acc, semA, semB,
                s, _NTILES, ngroups, rem)

        @pl.when(c == 0)
        def _():
            _edge_groups(hlo_hbm, *args)

        @pl.when(c == 1)
        def _():
            _edge_groups(hhi_hbm, *args)

        plsc.subcore_barrier()

        @pl.when(c == 0)
        def _():
            _row_chunks(s, lambda r0: pltpu.sync_copy(
                acc.at[pl.ds(r0, _ZROWS)],
                out_hbm.at[pl.ds(r0, _ZROWS), pl.ds(0, 128)]))

        @pl.when(c == 1)
        def _():
            _row_chunks(s, lambda r0: pltpu.sync_copy(
                acc.at[pl.ds(r0, _ZROWS)],
                out_hbm.at[pl.ds(r0, _ZROWS), pl.ds(128, 128)]))

    return k(h_lo, h_hi, src2, dst2)


def _agg_first(h0, src2, dst2):
    ngroups, rem = _EROWS // _GRP, _EROWS % _GRP

    @functools.partial(
        pl.kernel,
        out_type=jax.ShapeDtypeStruct((2, N, 128), jnp.float32),
        mesh=_MESH,
        scratch_types=_sc_scratch(),
    )
    def k(h_hbm, src_hbm, dst_hbm, out_hbm,
          sg, dg, rowsA, rowsB, zblk, acc, semA, semB):
        c = lax.axis_index("c")
        s = lax.axis_index("s")
        wid = c * _NTILES + s
        _zero_acc(zblk, acc, s)
        plsc.subcore_barrier()
        _edge_groups(h_hbm, src_hbm, dst_hbm, sg, dg, rowsA, rowsB, acc,
                     semA, semB, wid, 2 * _NTILES, ngroups, rem)
        plsc.subcore_barrier()
        _row_chunks(s, lambda r0: pltpu.sync_copy(
            acc.at[pl.ds(r0, _ZROWS)], out_hbm.at[c, pl.ds(r0, _ZROWS)]))

    return k(h0, src2, dst2)


_MROWS = 80
_MCHUNKS = N // _MROWS


def _segmax(hjk_lo, hjk_hi, batch):
    nw = 2 * _NTILES

    @functools.partial(
        pl.kernel,
        out_type=jax.ShapeDtypeStruct((nw, G, H), jnp.float32),
        mesh=_MESH,
        scratch_types=[
            pltpu.VMEM((_MROWS, 128), jnp.float32),
            pltpu.VMEM((_MROWS, 128), jnp.float32),
            pltpu.VMEM((_MROWS,), jnp.int32),
            pltpu.VMEM((G, H), jnp.float32),
        ],
    )
    def k(lo_hbm, hi_hbm, batch_hbm, out_hbm, rlo, rhi, bat, maxs):
        c = lax.axis_index("c")
        s = lax.axis_index("s")
        wid = c * _NTILES + s

        def mrow(r, _):
            for j in range(16):
                maxs[r, pl.ds(j * 16, 16)] = jnp.full((16,), -1e30, jnp.float32)
            return 0
        lax.fori_loop(0, G, mrow, 0)

        def do_chunk(cid):
            r0 = cid * _MROWS
            pltpu.sync_copy(lo_hbm.at[pl.ds(r0, _MROWS)], rlo)
            pltpu.sync_copy(hi_hbm.at[pl.ds(r0, _MROWS)], rhi)
            pltpu.sync_copy(batch_hbm.at[pl.ds(r0, _MROWS)], bat)

            def rowgrp(q, _):
                bv = bat[pl.ds(q * 16, 16)]
                for r in range(16):
                    g = bv[r]
                    rr = q * 16 + r
                    for j in range(8):
                        sl = pl.ds(j * 16, 16)
                        sh = pl.ds(j * 16 + 128, 16)
                        maxs[g, sl] = jnp.maximum(maxs[g, sl], rlo[rr, sl])
                        maxs[g, sh] = jnp.maximum(maxs[g, sh], rhi[rr, sl])
                return 0
            lax.fori_loop(0, _MROWS // 16, rowgrp, 0)

        for kk in range((_MCHUNKS + nw - 1) // nw):
            cid = wid + nw * kk

            @pl.when(cid < _MCHUNKS)
            def _():
                do_chunk(cid)

        pltpu.sync_copy(maxs, out_hbm.at[wid])

    return k(hjk_lo, hjk_hi, batch)


def _bn_in_body(x_ref, c_ref, b_ref, o_ref):
    o_ref[...] = x_ref[...] * c_ref[...] + b_ref[...]


def _bn_in(x, c, b):
    return pl.pallas_call(
        _bn_in_body,
        grid=(_GRID,),
        in_specs=[_rows(D), _full((1, D)), _full((1, D))],
        out_specs=_rows(D),
        out_shape=jax.ShapeDtypeStruct((N, D), jnp.float32),
    )(x, c[None, :], b[None, :])


def _mlp_body(h_ref, agg_ref, eps_ref, w1_ref, b1_ref, w2_ref, b2_ref,
              lo_ref, hi_ref):
    if agg_ref.shape[0] == 2:
        agg = agg_ref[0] + agg_ref[1]
    else:
        agg = agg_ref[...]
    u = (1.0 + eps_ref[0]) * h_ref[...] + agg
    z = jnp.dot(u, w1_ref[...], preferred_element_type=jnp.float32)
    z = jax.nn.gelu(z + b1_ref[...])
    z = jnp.dot(z, w2_ref[...], preferred_element_type=jnp.float32)
    z = jax.nn.gelu(z + b2_ref[...])
    lo_ref[...] = z[:, :128]
    hi_ref[...] = z[:, 128:]


def _mlp(h, agg, eps, w1, b1, w2, b2):
    hin = h.shape[1]
    if agg.ndim == 3:
        agg_spec = pl.BlockSpec((2, _BLK, 128), lambda i: (0, i, 0))
    else:
        agg_spec = _rows(hin)
    return pl.pallas_call(
        _mlp_body,
        grid=(_GRID,),
        in_specs=[_rows(hin), agg_spec, _full((1,)), _full((hin, H)),
                  _full((1, H)), _full((H, H)), _full((1, H))],
        out_specs=[_rows(128), _rows(128)],
        out_shape=[jax.ShapeDtypeStruct((N, 128), jnp.float32),
                   jax.ShapeDtypeStruct((N, 128), jnp.float32)],
    )(h, agg, eps[None], w1, b1[None, :], w2, b2[None, :])


def _jk_body(h1l, h1h, h2l, h2h, h3l, h3h, batch_ref, attT_ref,
             lo_ref, hi_ref, sums_ref, cnts_ref):
    i = pl.program_id(0)

    @pl.when(i == 0)
    def _init():
        sums_ref[...] = jnp.zeros_like(sums_ref)
        cnts_ref[...] = jnp.zeros_like(cnts_ref)

    h1 = jnp.concatenate([h1l[...], h1h[...]], axis=1)
    h2 = jnp.concatenate([h2l[...], h2h[...]], axis=1)
    h3 = jnp.concatenate([h3l[...], h3h[...]], axis=1)
    attT = attT_ref[...]
    s1 = jnp.dot(h1, attT[:, 0:1], preferred_element_type=jnp.float32)
    s2 = jnp.dot(h2, attT[:, 1:2], preferred_element_type=jnp.float32)
    s3 = jnp.dot(h3, attT[:, 2:3], preferred_element_type=jnp.float32)
    m = jnp.maximum(jnp.maximum(s1, s2), s3)
    e1 = jnp.exp(s1 - m)
    e2 = jnp.exp(s2 - m)
    e3 = jnp.exp(s3 - m)
    inv = 1.0 / (e1 + e2 + e3)
    hjk = (h1 * e1 + h2 * e2 + h3 * e3) * inv
    lo_ref[...] = hjk[:, :128]
    hi_ref[...] = hjk[:, 128:]

    b = batch_ref[0, 0, :]
    gids = jax.lax.broadcasted_iota(jnp.int32, (_BLK, G), 1)
    maskf = (b[:, None] == gids).astype(jnp.float32)
    dn = (((0,), (0,)), ((), ()))
    sums_ref[...] += jax.lax.dot_general(maskf, hjk, dn,
                                         preferred_element_type=jnp.float32)
    cnts_ref[...] += jax.lax.dot_general(
        maskf, jnp.ones((_BLK, 128), jnp.float32), dn,
        preferred_element_type=jnp.float32)


def _jk(parts, batch3, attT):
    (h1l, h1h), (h2l, h2h), (h3l, h3h) = parts
    return pl.pallas_call(
        _jk_body,
        grid=(_GRID,),
        in_specs=[_rows(128)] * 6 + [
            pl.BlockSpec((1, 1, _BLK), lambda i: (i, 0, 0)),
            _full((H, 3))],
        out_specs=[_rows(128), _rows(128), _full((G, H)), _full((G, 128))],
        out_shape=[jax.ShapeDtypeStruct((N, 128), jnp.float32),
                   jax.ShapeDtypeStruct((N, 128), jnp.float32),
                   jax.ShapeDtypeStruct((G, H), jnp.float32),
                   jax.ShapeDtypeStruct((G, 128), jnp.float32)],
    )(h1l, h1h, h2l, h2h, h3l, h3h, batch3, attT)


def _head_body(sums_ref, cnts_ref, mxp_ref, poolw_ref, w1_ref, b1_ref,
               lng_ref, lnb_ref, w2_ref, b2_ref, out_ref):
    sums = sums_ref[...]
    cnt = cnts_ref[:, 0:1]
    mean = sums / jnp.maximum(cnt, 1.0)
    mx = mxp_ref[pl.ds(0, G), :]
    for j in range(1, 2 * _NTILES):
        mx = jnp.maximum(mx, mxp_ref[pl.ds(j * G, G), :])
    mx = jnp.where(cnt > 0.0, mx, 0.0)
    pw = jax.nn.softmax(poolw_ref[...], axis=1)
    pooled = sums * pw[:, 0:1] + mean * pw[:, 1:2] + mx * pw[:, 2:3]
    z = jnp.dot(pooled, w1_ref[...], preferred_element_type=jnp.float32)
    z = z + b1_ref[...]
    mu_ = jnp.mean(z, axis=-1, keepdims=True)
    var_ = jnp.mean((z - mu_) ** 2, axis=-1, keepdims=True)
    z = (z - mu_) * jax.lax.rsqrt(var_ + 1e-5) * lng_ref[...] + lnb_ref[...]
    z = jax.nn.gelu(z) + pooled
    out = jnp.dot(z, w2_ref[...], preferred_element_type=jnp.float32)
    out_ref[...] = out + b2_ref[...]


def _head(sums, cnts, mxp, pool_w, p):
    return pl.pallas_call(
        _head_body,
        out_shape=jax.ShapeDtypeStruct((G, 2 * LAT), jnp.float32),
    )(sums, cnts, mxp, pool_w, p['fc1_w'], p['fc1_b'][None, :],
      p['ln_g'][None, :], p['ln_b'][None, :], p['fc2_w'], p['fc2_b'][None, :])


def kernel(x, edge_index, batch, params):
    src = edge_index[0].reshape(_EROWS, _CHUNK)
    dst = edge_index[1].reshape(_EROWS, _CHUNK)
    p = params

    h0 = _bn_in(x, _BN_SCALE * p['in_bn_g'], p['in_bn_b'])

    hs = []
    h = h0
    for l in range(L):
        c = p['convs'][l]
        s1 = _BN_SCALE * c['bn1_g']
        w1 = c['fc1_w'] * s1[None, :]
        b1 = c['fc1_b'] * s1 + c['bn1_b']
        s2 = _BN_SCALE * p['bns_g'][l]
        w2 = c['fc2_w'] * s2[None, :]
        b2 = c['fc2_b'] * s2 + p['bns_b'][l]
        if l == 0:
            agg = _agg_first(h, src, dst)
        else:
            agg = _agg_half(hs[-1][0], hs[-1][1], src, dst)
        parts = _mlp(h, agg, c['eps'], w1, b1, w2, b2)
        hs.append(parts)
        h = jnp.concatenate(parts, axis=1)

    batch3 = batch.reshape(_GRID, 1, _BLK)
    attT = (p['att_w'] / H).T
    hjk_lo, hjk_hi, sums, cnts = _jk(hs, batch3, attT)

    mxp = _segmax(hjk_lo, hjk_hi, batch).reshape(2 * _NTILES * G, H)
    out = _head(sums, cnts, mxp, p['pool_w'][None, :], p)
    mu, logvar = jnp.split(out, 2, axis=-1)
    return (mu, logvar)


# speedup vs baseline: 7.9253x; 1.0807x over previous
import functools

import jax
import jax.numpy as jnp
from jax import lax
from jax.experimental import pallas as pl
from jax.experimental.pallas import tpu as pltpu
from jax.experimental.pallas import tpu_sc as plsc

N = 10000
E = 320000
D = 128
H = 256
LAT = 64
L = 3
G = 64

_BN_SCALE = 1.0 / (1.0 + 1e-5) ** 0.5
_BLK = 1000
_GRID = N // _BLK


def _full(shape):
    return pl.BlockSpec(shape, lambda i: tuple(0 for _ in shape))


def _rows(width):
    return pl.BlockSpec((_BLK, width), lambda i: (i, 0))


_NTILES = 16
_CHUNK = 128
_ZROWS = 200
_NCHUNKS = N // _ZROWS


_ZB = 40


def _zero_block(zblk):
    def zrow(r, _):
        for j in range(8):
            zblk[r, pl.ds(j * 16, 16)] = jnp.zeros((16,), jnp.float32)
        return 0
    lax.fori_loop(0, _ZB, zrow, 0)


def _row_chunks(s, fn):
    for kk in range((_NCHUNKS + _NTILES - 1) // _NTILES):
        j = s + _NTILES * kk

        @pl.when(j < _NCHUNKS)
        def _():
            fn(j * _ZROWS)


def _zero_acc(zblk, acc, s):
    _zero_block(zblk)

    def zchunk(r0):
        for t in range(_ZROWS // _ZB):
            pltpu.sync_copy(zblk, acc.at[pl.ds(r0 + t * _ZB, _ZB)])
    _row_chunks(s, zchunk)


_GRP = 16
_EROWS = E // _CHUNK


def _edge_groups(table, src2, dst2, sg, dg, rowsA, rowsB, acc, semA, semB,
                 worker, nworkers, ngroups, rem):
    def do_group(r0):
        pltpu.sync_copy(src2.at[pl.ds(r0, _GRP)], sg)
        pltpu.sync_copy(dst2.at[pl.ds(r0, _GRP)], dg)
        pltpu.async_copy(table.at[sg.at[0]], rowsA, semA)
        for q in range(_GRP // 2):
            pltpu.async_copy(table.at[sg.at[2 * q + 1]], rowsB, semB)
            pltpu.make_async_copy(table.at[sg.at[0]], rowsA, semA).wait()
            pltpu.sync_copy(rowsA, acc.at[dg.at[2 * q]], add=True)
            if q < _GRP // 2 - 1:
                pltpu.async_copy(table.at[sg.at[2 * q + 2]], rowsA, semA)
            pltpu.make_async_copy(table.at[sg.at[0]], rowsB, semB).wait()
            pltpu.sync_copy(rowsB, acc.at[dg.at[2 * q + 1]], add=True)

    def body(gi, _):
        j = worker + nworkers * gi

        @pl.when(j < ngroups)
        def _():
            do_group(pl.multiple_of(j * _GRP, _GRP))
        return 0
    lax.fori_loop(0, (ngroups + nworkers - 1) // nworkers, body, 0)

    if rem:
        @pl.when(worker == 0)
        def _():
            r0 = ngroups * _GRP
            pltpu.sync_copy(src2.at[pl.ds(r0, rem)], sg.at[pl.ds(0, rem)])
            pltpu.sync_copy(dst2.at[pl.ds(r0, rem)], dg.at[pl.ds(0, rem)])
            for q in range(rem):
                pltpu.async_copy(table.at[sg.at[q]], rowsA, semA).wait()
                pltpu.sync_copy(rowsA, acc.at[dg.at[q]], add=True)


def _sc_scratch():
    return [
        pltpu.VMEM((_GRP, _CHUNK), jnp.int32),
        pltpu.VMEM((_GRP, _CHUNK), jnp.int32),
        pltpu.VMEM((_CHUNK, 128), jnp.float32),
        pltpu.VMEM((_CHUNK, 128), jnp.float32),
        pltpu.VMEM((_ZB, 128), jnp.float32),
        pltpu.VMEM_SHARED((N, 128), jnp.float32),
        pltpu.SemaphoreType.DMA,
        pltpu.SemaphoreType.DMA,
    ]
_MESH = plsc.VectorSubcoreMesh(core_axis_name="c", subcore_axis_name="s")


def _agg_half(h_lo, h_hi, src2, dst2):
    ngroups, rem = _EROWS // _GRP, _EROWS % _GRP

    @functools.partial(
        pl.kernel,
        out_type=jax.ShapeDtypeStruct((N, H), jnp.float32),
        mesh=_MESH,
        scratch_types=_sc_scratch(),
    )
    def k(hlo_hbm, hhi_hbm, src_hbm, dst_hbm, out_hbm,
          sg, dg, rowsA, rowsB, zblk, acc, semA, semB):
        c = lax.axis_index("c")
        s = lax.axis_index("s")
        _zero_acc(zblk, acc, s)
        plsc.subcore_barrier()
        args = (src_hbm, dst_hbm, sg, dg, rowsA, rowsB, acc, semA, semB,
                s, _NTILES, ngroups, rem)

        @pl.when(c == 0)
        def _():
            _edge_groups(hlo_hbm, *args)

        @pl.when(c == 1)
        def _():
            _edge_groups(hhi_hbm, *args)

        plsc.subcore_barrier()

        @pl.when(c == 0)
        def _():
            _row_chunks(s, lambda r0: pltpu.sync_copy(
                acc.at[pl.ds(r0, _ZROWS)],
                out_hbm.at[pl.ds(r0, _ZROWS), pl.ds(0, 128)]))

        @pl.when(c == 1)
        def _():
            _row_chunks(s, lambda r0: pltpu.sync_copy(
                acc.at[pl.ds(r0, _ZROWS)],
                out_hbm.at[pl.ds(r0, _ZROWS), pl.ds(128, 128)]))

    return k(h_lo, h_hi, src2, dst2)


def _agg_first(h0, src2, dst2):
    ngroups, rem = _EROWS // _GRP, _EROWS % _GRP

    @functools.partial(
        pl.kernel,
        out_type=jax.ShapeDtypeStruct((2, N, 128), jnp.float32),
        mesh=_MESH,
        scratch_types=_sc_scratch(),
    )
    def k(h_hbm, src_hbm, dst_hbm, out_hbm,
          sg, dg, rowsA, rowsB, zblk, acc, semA, semB):
        c = lax.axis_index("c")
        s = lax.axis_index("s")
        wid = c * _NTILES + s
        _zero_acc(zblk, acc, s)
        plsc.subcore_barrier()
        _edge_groups(h_hbm, src_hbm, dst_hbm, sg, dg, rowsA, rowsB, acc,
                     semA, semB, wid, 2 * _NTILES, ngroups, rem)
        plsc.subcore_barrier()
        _row_chunks(s, lambda r0: pltpu.sync_copy(
            acc.at[pl.ds(r0, _ZROWS)], out_hbm.at[c, pl.ds(r0, _ZROWS)]))

    return k(h0, src2, dst2)


_MROWS = 80
_MCHUNKS = N // _MROWS


def _segmax(hjk_lo, hjk_hi, batch):
    nw = 2 * _NTILES

    @functools.partial(
        pl.kernel,
        out_type=jax.ShapeDtypeStruct((nw, G, H), jnp.float32),
        mesh=_MESH,
        scratch_types=[
            pltpu.VMEM((_MROWS, 128), jnp.float32),
            pltpu.VMEM((_MROWS, 128), jnp.float32),
            pltpu.VMEM((_MROWS,), jnp.int32),
            pltpu.VMEM((G, H), jnp.float32),
        ],
    )
    def k(lo_hbm, hi_hbm, batch_hbm, out_hbm, rlo, rhi, bat, maxs):
        c = lax.axis_index("c")
        s = lax.axis_index("s")
        wid = c * _NTILES + s

        def mrow(r, _):
            for j in range(16):
                maxs[r, pl.ds(j * 16, 16)] = jnp.full((16,), -1e30, jnp.float32)
            return 0
        lax.fori_loop(0, G, mrow, 0)

        def do_chunk(cid):
            r0 = cid * _MROWS
            pltpu.sync_copy(lo_hbm.at[pl.ds(r0, _MROWS)], rlo)
            pltpu.sync_copy(hi_hbm.at[pl.ds(r0, _MROWS)], rhi)
            pltpu.sync_copy(batch_hbm.at[pl.ds(r0, _MROWS)], bat)

            def rowgrp(q, _):
                bv = bat[pl.ds(q * 16, 16)]
                for r in range(16):
                    g = bv[r]
                    rr = q * 16 + r
                    for j in range(8):
                        sl = pl.ds(j * 16, 16)
                        sh = pl.ds(j * 16 + 128, 16)
                        maxs[g, sl] = jnp.maximum(maxs[g, sl], rlo[rr, sl])
                        maxs[g, sh] = jnp.maximum(maxs[g, sh], rhi[rr, sl])
                return 0
            lax.fori_loop(0, _MROWS // 16, rowgrp, 0)

        for kk in range((_MCHUNKS + nw - 1) // nw):
            cid = wid + nw * kk

            @pl.when(cid < _MCHUNKS)
            def _():
                do_chunk(cid)

        pltpu.sync_copy(maxs, out_hbm.at[wid])

    return k(hjk_lo, hjk_hi, batch)


def _bn_in_body(x_ref, c_ref, b_ref, o_ref):
    o_ref[...] = x_ref[...] * c_ref[...] + b_ref[...]


def _bn_in(x, c, b):
    return pl.pallas_call(
        _bn_in_body,
        grid=(_GRID,),
        in_specs=[_rows(D), _full((1, D)), _full((1, D))],
        out_specs=_rows(D),
        out_shape=jax.ShapeDtypeStruct((N, D), jnp.float32),
    )(x, c[None, :], b[None, :])


def _mlp_body(h_ref, agg_ref, eps_ref, w1_ref, b1_ref, w2_ref, b2_ref,
              lo_ref, hi_ref):
    if agg_ref.shape[0] == 2:
        agg = agg_ref[0] + agg_ref[1]
    else:
        agg = agg_ref[...]
    u = (1.0 + eps_ref[0]) * h_ref[...] + agg
    z = jnp.dot(u, w1_ref[...], preferred_element_type=jnp.float32)
    z = jax.nn.gelu(z + b1_ref[...])
    z = jnp.dot(z, w2_ref[...], preferred_element_type=jnp.float32)
    z = jax.nn.gelu(z + b2_ref[...])
    lo_ref[...] = z[:, :128]
    hi_ref[...] = z[:, 128:]


def _mlp(h, agg, eps, w1, b1, w2, b2):
    hin = h.shape[1]
    if agg.ndim == 3:
        agg_spec = pl.BlockSpec((2, _BLK, 128), lambda i: (0, i, 0))
    else:
        agg_spec = _rows(hin)
    return pl.pallas_call(
        _mlp_body,
        grid=(_GRID,),
        in_specs=[_rows(hin), agg_spec, _full((1,)), _full((hin, H)),
                  _full((1, H)), _full((H, H)), _full((1, H))],
        out_specs=[_rows(128), _rows(128)],
        out_shape=[jax.ShapeDtypeStruct((N, 128), jnp.float32),
                   jax.ShapeDtypeStruct((N, 128), jnp.float32)],
    )(h, agg, eps[None], w1, b1[None, :], w2, b2[None, :])


def _jk_body(h1l, h1h, h2l, h2h, h3l, h3h, batch_ref, attT_ref,
             lo_ref, hi_ref, sums_ref, cnts_ref):
    i = pl.program_id(0)

    @pl.when(i == 0)
    def _init():
        sums_ref[...] = jnp.zeros_like(sums_ref)
        cnts_ref[...] = jnp.zeros_like(cnts_ref)

    h1 = jnp.concatenate([h1l[...], h1h[...]], axis=1)
    h2 = jnp.concatenate([h2l[...], h2h[...]], axis=1)
    h3 = jnp.concatenate([h3l[...], h3h[...]], axis=1)
    attT = attT_ref[...]
    s1 = jnp.dot(h1, attT[:, 0:1], preferred_element_type=jnp.float32)
    s2 = jnp.dot(h2, attT[:, 1:2], preferred_element_type=jnp.float32)
    s3 = jnp.dot(h3, attT[:, 2:3], preferred_element_type=jnp.float32)
    m = jnp.maximum(jnp.maximum(s1, s2), s3)
    e1 = jnp.exp(s1 - m)
    e2 = jnp.exp(s2 - m)
    e3 = jnp.exp(s3 - m)
    inv = 1.0 / (e1 + e2 + e3)
    hjk = (h1 * e1 + h2 * e2 + h3 * e3) * inv
    lo_ref[...] = hjk[:, :128]
    hi_ref[...] = hjk[:, 128:]

    b = batch_ref[0, 0, :]
    gids = jax.lax.broadcasted_iota(jnp.int32, (_BLK, G), 1)
    maskf = (b[:, None] == gids).astype(jnp.float32)
    dn = (((0,), (0,)), ((), ()))
    sums_ref[...] += jax.lax.dot_general(maskf, hjk, dn,
                                         preferred_element_type=jnp.float32)
    cnts_ref[...] += jax.lax.dot_general(
        maskf, jnp.ones((_BLK, 128), jnp.float32), dn,
        preferred_element_type=jnp.float32)


def _jk(parts, batch3, attT):
    (h1l, h1h), (h2l, h2h), (h3l, h3h) = parts
    return pl.pallas_call(
        _jk_body,
        grid=(_GRID,),
        in_specs=[_rows(128)] * 6 + [
            pl.BlockSpec((1, 1, _BLK), lambda i: (i, 0, 0)),
            _full((H, 3))],
        out_specs=[_rows(128), _rows(128), _full((G, H)), _full((G, 128))],
        out_shape=[jax.ShapeDtypeStruct((N, 128), jnp.float32),
                   jax.ShapeDtypeStruct((N, 128), jnp.float32),
                   jax.ShapeDtypeStruct((G, H), jnp.float32),
                   jax.ShapeDtypeStruct((G, 128), jnp.float32)],
    )(h1l, h1h, h2l, h2h, h3l, h3h, batch3, attT)


def _head_body(sums_ref, cnts_ref, mxp_ref, poolw_ref, w1_ref, b1_ref,
               lng_ref, lnb_ref, w2_ref, b2_ref, out_ref):
    sums = sums_ref[...]
    cnt = cnts_ref[:, 0:1]
    mean = sums / jnp.maximum(cnt, 1.0)
    mx = mxp_ref[pl.ds(0, G), :]
    for j in range(1, 2 * _NTILES):
        mx = jnp.maximum(mx, mxp_ref[pl.ds(j * G, G), :])
    mx = jnp.where(cnt > 0.0, mx, 0.0)
    pw = jax.nn.softmax(poolw_ref[...], axis=1)
    pooled = sums * pw[:, 0:1] + mean * pw[:, 1:2] + mx * pw[:, 2:3]
    z = jnp.dot(pooled, w1_ref[...], preferred_element_type=jnp.float32)
    z = z + b1_ref[...]
    mu_ = jnp.mean(z, axis=-1, keepdims=True)
    var_ = jnp.mean((z - mu_) ** 2, axis=-1, keepdims=True)
    z = (z - mu_) * jax.lax.rsqrt(var_ + 1e-5) * lng_ref[...] + lnb_ref[...]
    z = jax.nn.gelu(z) + pooled
    out = jnp.dot(z, w2_ref[...], preferred_element_type=jnp.float32)
    out_ref[...] = out + b2_ref[...]


def _head(sums, cnts, mxp, pool_w, p):
    return pl.pallas_call(
        _head_body,
        out_shape=jax.ShapeDtypeStruct((G, 2 * LAT), jnp.float32),
    )(sums, cnts, mxp, pool_w, p['fc1_w'], p['fc1_b'][None, :],
      p['ln_g'][None, :], p['ln_b'][None, :], p['fc2_w'], p['fc2_b'][None, :])


def kernel(x, edge_index, batch, params):
    src = edge_index[0].reshape(_EROWS, _CHUNK)
    dst = edge_index[1].reshape(_EROWS, _CHUNK)
    p = params

    h0 = _bn_in(x, _BN_SCALE * p['in_bn_g'], p['in_bn_b'])

    hs = []
    h = h0
    for l in range(L):
        c = p['convs'][l]
        s1 = _BN_SCALE * c['bn1_g']
        w1 = c['fc1_w'] * s1[None, :]
        b1 = c['fc1_b'] * s1 + c['bn1_b']
        s2 = _BN_SCALE * p['bns_g'][l]
        w2 = c['fc2_w'] * s2[None, :]
        b2 = c['fc2_b'] * s2 + p['bns_b'][l]
        if l == 0:
            agg = _agg_first(h, src, dst)
        else:
            agg = _agg_half(hs[-1][0], hs[-1][1], src, dst)
        parts = _mlp(h, agg, c['eps'], w1, b1, w2, b2)
        hs.append(parts)
        h = jnp.concatenate(parts, axis=1)

    batch3 = batch.reshape(_GRID, 1, _BLK)
    attT = (p['att_w'] / H).T
    hjk_lo, hjk_hi, sums, cnts = _jk(hs, batch3, attT)

    mxp = _segmax(hjk_lo, hjk_hi, batch).reshape(2 * _NTILES * G, H)
    out = _head(sums, cnts, mxp, p['pool_w'][None, :], p)
    mu, logvar = jnp.split(out, 2, axis=-1)
    return (mu, logvar)
